# Initial kernel scaffold; baseline (speedup 1.0000x reference)
import jax
import jax.numpy as jnp
from jax.experimental import pallas as pl


def kernel(signal, W, coord0, coord1):
    raise NotImplementedError("write your pallas kernel here")


# speedup vs baseline: 3.4607x; 3.4607x over previous
import functools
import math

import jax
import jax.numpy as jnp
from jax import lax
from jax.experimental import pallas as pl
from jax.experimental.pallas import tpu as pltpu
from jax.experimental.pallas import tpu_sc as plsc

_SZ = 256
_A = 4
_Z = 4
_K = 9
_NITEMS = _K * _A * _SZ * _SZ
_ROWS = _NITEMS // 128
_NW = 32
_ROWS_PER_W = _ROWS // _NW
_CHUNK_ROWS = 8
_NCHUNK = _ROWS_PER_W // _CHUNK_ROWS
_HALF = (_SZ * _SZ) // 2
_R = 64


def _phase1_body(s_ref, b_ref, a_ref, w_ref, sig_ref, vals_ref, idx_ref):
    wmat = w_ref[0]
    sig = sig_ref[...]
    dec = [[None] * 3 for _ in range(2)]
    for dim in range(2):
        for di in range(3):
            s = s_ref[dim, di, 0]
            b = b_ref[dim, di, 0]
            a = a_ref[dim, di, 0]
            cc = jnp.sqrt((1.0 - s * s) + 1e-6)
            f = a * b / cc
            f = 2.0 * f / math.pi
            f = (f + 1.0) / 2.0
            dec[dim][di] = f * 255.0
    r = [[jnp.round(dec[dim][di]).astype(jnp.int32) for di in range(3)]
         for dim in range(2)]
    e = [[(r[dim][di].astype(jnp.float32) - dec[dim][1]) ** 2
          for di in range(3)] for dim in range(2)]
    for d0 in range(3):
        for d1 in range(3):
            k = d0 * 3 + d1
            src = jnp.sqrt(e[0][d0] + e[1][d1])
            srcv = 6.0 * (1.0 - 2.0 * src)
            wk = 1.0 / (1.0 + jnp.exp(-srcv))
            coeff = wmat * wk
            cil = jnp.broadcast_to(coeff[:, :, None],
                                   (_R, _SZ, _Z)).reshape(_R, _SZ * _Z)
            vals_ref[k, 0] = cil * sig
            ii = r[0][d0] * _SZ + r[1][d1]
            ii = jnp.clip(ii, 0, _SZ * _SZ - 1)
            idx_ref[0, k, 0] = jnp.where(ii < _HALF, ii, -1)
            idx_ref[1, k, 0] = jnp.where(ii >= _HALF, ii - _HALF, -1)


def _phase1(sin_a, cos_a, asn_a, w, sig_il):
    nb = _SZ // _R
    grid = (_A, nb)
    ang_spec = pl.BlockSpec((2, 3, 1, _R, _SZ), lambda a, rb: (0, 0, a, rb, 0))
    return pl.pallas_call(
        _phase1_body,
        grid=grid,
        in_specs=[
            ang_spec, ang_spec, ang_spec,
            pl.BlockSpec((1, _R, _SZ), lambda a, rb: (a, rb, 0)),
            pl.BlockSpec((_R, _SZ * _Z), lambda a, rb: (rb, 0)),
        ],
        out_specs=[
            pl.BlockSpec((_K, 1, _R, _SZ * _Z), lambda a, rb: (0, a, rb, 0)),
            pl.BlockSpec((2, _K, 1, _R, _SZ), lambda a, rb: (0, 0, a, rb, 0)),
        ],
        out_shape=[
            jax.ShapeDtypeStruct((_K, _A, _SZ, _SZ * _Z), jnp.float32),
            jax.ShapeDtypeStruct((2, _K, _A, _SZ, _SZ), jnp.int32),
        ],
    )(sin_a, cos_a, asn_a, w, sig_il)


def _phase2(idx_r, vals_r, zeros):
    mesh = plsc.VectorSubcoreMesh(core_axis_name="c", subcore_axis_name="s")

    @functools.partial(
        pl.kernel,
        out_type=jax.ShapeDtypeStruct((2, 2, _HALF, _Z), jnp.float32),
        mesh=mesh,
        compiler_params=pltpu.CompilerParams(use_tc_tiling_on_sc=False),
        scratch_types=[
            pltpu.VMEM((2, _CHUNK_ROWS, 128), jnp.int32),
            pltpu.VMEM((_CHUNK_ROWS, 128, _Z), jnp.float32),
            (pltpu.VMEM_SHARED @ mesh)((_HALF, _Z), jnp.float32),
            (pltpu.VMEM_SHARED @ mesh)((_HALF, _Z), jnp.float32),
        ],
    )
    def run(idx_h, vals_h, z_h, out_h, idxbuf, valbuf, acc_lo, acc_hi):
        ci = lax.axis_index("c")
        si = lax.axis_index("s")
        wid = si * 2 + ci
        seg = _HALF // 16
        pltpu.sync_copy(z_h.at[pl.ds(si * seg, seg)],
                        acc_lo.at[pl.ds(si * seg, seg)])
        pltpu.sync_copy(z_h.at[pl.ds(si * seg, seg)],
                        acc_hi.at[pl.ds(si * seg, seg)])
        plsc.subcore_barrier()
        row0 = wid * _ROWS_PER_W

        def chunk(it, carry):
            rr = row0 + it * _CHUNK_ROWS
            pltpu.sync_copy(idx_h.at[0, pl.ds(rr, _CHUNK_ROWS)], idxbuf.at[0])
            pltpu.sync_copy(idx_h.at[1, pl.ds(rr, _CHUNK_ROWS)], idxbuf.at[1])
            pltpu.sync_copy(vals_h.at[pl.ds(rr, _CHUNK_ROWS)], valbuf)
            for j in range(_CHUNK_ROWS):
                lo = plsc.Indices(idxbuf.at[0, j], ignored_value=-1)
                hi = plsc.Indices(idxbuf.at[1, j], ignored_value=-1)
                pltpu.sync_copy(valbuf.at[j], acc_lo.at[lo], add=True)
                pltpu.sync_copy(valbuf.at[j], acc_hi.at[hi], add=True)
            return carry

        lax.fori_loop(0, _NCHUNK, chunk, 0)
        plsc.subcore_barrier()
        pltpu.sync_copy(acc_lo.at[pl.ds(si * seg, seg)],
                        out_h.at[ci, 0, pl.ds(si * seg, seg)])
        pltpu.sync_copy(acc_hi.at[pl.ds(si * seg, seg)],
                        out_h.at[ci, 1, pl.ds(si * seg, seg)])

    return run(idx_r, vals_r, zeros)


def _phase3_body(p_ref, o_ref):
    o_ref[...] = p_ref[0, 0] + p_ref[1, 0]


def _phase3(partial):
    rb = 4096
    nb = _HALF // rb
    return pl.pallas_call(
        _phase3_body,
        grid=(2, nb),
        in_specs=[pl.BlockSpec((2, 1, rb, _Z), lambda h, i: (0, h, i, 0))],
        out_specs=pl.BlockSpec((rb, _Z), lambda h, i: (h * 8 + i, 0)),
        out_shape=jax.ShapeDtypeStruct((_SZ * _SZ, _Z), jnp.float32),
    )(partial)


def kernel(signal, W, coord0, coord1):
    sh = jnp.array([math.pi * d / (_SZ - 1) for d in (-1, 0, 1)],
                   dtype=jnp.float32)
    th0 = coord0[None] + sh[:, None, None, None]
    th1 = coord1[None] + sh[:, None, None, None]
    th = jnp.stack([th0, th1], axis=0)
    sin_a = jnp.sin(th)
    cos_a = jnp.cos(th)
    asn_a = jnp.arcsin(sin_a)
    sig_il = signal.transpose(1, 2, 0).reshape(_SZ, _SZ * _Z)

    vals, idx = _phase1(sin_a, cos_a, asn_a, W, sig_il)
    idx_r = idx.reshape(2, _ROWS, 128)
    vals_r = vals.reshape(_ROWS, 128, _Z)
    zeros = jnp.zeros((_HALF, _Z), jnp.float32)
    part = _phase2(idx_r, vals_r, zeros)
    comb = _phase3(part)
    return comb.T.reshape(_Z, _SZ, _SZ)
